# baseline (speedup 1.0000x reference)
import jax
import jax.numpy as jnp
from jax.experimental import pallas as pl

NUM_EXPERTS = 64
D_MODEL = 2048
BLOCK_T = 512


def _router_body(x_ref, w_ref, oh_ref, mp_ref, lg_ref):
    x = x_ref[...]
    w = w_ref[...]
    logits = jax.lax.dot_general(
        w, x, (((1,), (1,)), ((), ())),
        preferred_element_type=jnp.float32)
    m = jnp.max(logits, axis=0, keepdims=True)
    e = jnp.exp(logits - m)
    s = jnp.sum(e, axis=0, keepdims=True)
    probs = e / s
    mp = jnp.max(probs, axis=0, keepdims=True)
    row = jax.lax.broadcasted_iota(jnp.int32, probs.shape, 0)
    idx = jnp.min(jnp.where(probs == mp, row, NUM_EXPERTS),
                  axis=0, keepdims=True)
    oh_ref[...] = (row == idx).astype(jnp.int32)
    mp_ref[...] = mp
    lg_ref[...] = logits


def kernel(x, W):
    n = x.shape[0]
    oh_t, mp_t, lg_t = pl.pallas_call(
        _router_body,
        grid=(n // BLOCK_T,),
        in_specs=[
            pl.BlockSpec((BLOCK_T, D_MODEL), lambda i: (i, 0)),
            pl.BlockSpec((NUM_EXPERTS, D_MODEL), lambda i: (0, 0)),
        ],
        out_specs=[
            pl.BlockSpec((NUM_EXPERTS, BLOCK_T), lambda i: (0, i)),
            pl.BlockSpec((1, BLOCK_T), lambda i: (0, i)),
            pl.BlockSpec((NUM_EXPERTS, BLOCK_T), lambda i: (0, i)),
        ],
        out_shape=[
            jax.ShapeDtypeStruct((NUM_EXPERTS, n), jnp.int32),
            jax.ShapeDtypeStruct((1, n), jnp.float32),
            jax.ShapeDtypeStruct((NUM_EXPERTS, n), jnp.float32),
        ],
    )(x, W)
    return oh_t.T, mp_t.T, lg_t.T


# speedup vs baseline: 1.1678x; 1.1678x over previous
import jax
import jax.numpy as jnp
from jax.experimental import pallas as pl

NUM_EXPERTS = 64
D_MODEL = 2048
BLOCK_T = 1024


def _router_body(x_ref, w_ref, oh_ref, mp_ref, lg_ref):
    x = x_ref[...]
    w = w_ref[...]
    logits = jax.lax.dot_general(
        w, x, (((1,), (1,)), ((), ())),
        preferred_element_type=jnp.float32)
    m = jnp.max(logits, axis=0, keepdims=True)
    e = jnp.exp(logits - m)
    s = jnp.sum(e, axis=0, keepdims=True)
    probs = e / s
    mp = jnp.max(probs, axis=0, keepdims=True)
    row = jax.lax.broadcasted_iota(jnp.int32, probs.shape, 0)
    idx = jnp.min(jnp.where(probs == mp, row, NUM_EXPERTS),
                  axis=0, keepdims=True)
    oh_ref[...] = (row == idx).astype(jnp.int32)
    mp_ref[...] = mp
    lg_ref[...] = logits


def kernel(x, W):
    n = x.shape[0]
    oh_t, mp_t, lg_t = pl.pallas_call(
        _router_body,
        grid=(n // BLOCK_T,),
        in_specs=[
            pl.BlockSpec((BLOCK_T, D_MODEL), lambda i: (i, 0)),
            pl.BlockSpec((NUM_EXPERTS, D_MODEL), lambda i: (0, 0)),
        ],
        out_specs=[
            pl.BlockSpec((NUM_EXPERTS, BLOCK_T), lambda i: (0, i)),
            pl.BlockSpec((1, BLOCK_T), lambda i: (0, i)),
            pl.BlockSpec((NUM_EXPERTS, BLOCK_T), lambda i: (0, i)),
        ],
        out_shape=[
            jax.ShapeDtypeStruct((NUM_EXPERTS, n), jnp.int32),
            jax.ShapeDtypeStruct((1, n), jnp.float32),
            jax.ShapeDtypeStruct((NUM_EXPERTS, n), jnp.float32),
        ],
    )(x, W)
    return oh_t.T, mp_t.T, lg_t.T
